# Initial kernel scaffold; baseline (speedup 1.0000x reference)
import jax
import jax.numpy as jnp
from jax.experimental import pallas as pl


def kernel(ChannelNoiseMatixs, targets):
    raise NotImplementedError("write your pallas kernel here")


# speedup vs baseline: 1.0352x; 1.0352x over previous
import jax
import jax.numpy as jnp
from jax.experimental import pallas as pl
from jax.experimental.pallas import tpu as pltpu

N_MATS = 4
B = 16384
D = 512
NUM_CLASSES = 10
ROWS = 2048
NB = B // ROWS
NSTEPS = N_MATS * NB
INV_LN2 = 1.4426950408889634


def _body(t_full_ref, t_col_ref, x_ref, out_ref, inv_ref, acc_ref):
    i = pl.program_id(0)
    j = pl.program_id(1)
    step = i * NB + j

    @pl.when(step == 0)
    def _init():
        t_full = t_full_ref[...]
        counts = []
        for c in range(NUM_CLASSES):
            counts.append(jnp.sum((t_full == c).astype(jnp.float32)))
        c_present = sum(
            [jnp.where(cnt > 0, 1.0, 0.0) for cnt in counts], 0.0
        )
        scale = 1.0 / (4.0 * c_present)
        for c in range(NUM_CLASSES):
            inv_ref[c] = jnp.where(
                counts[c] > 0, scale / counts[c], 0.0
            )
        acc_ref[0] = 0.0

    x = x_ref[0]
    s2 = jnp.sum(x * x, axis=1, keepdims=True)
    inv_norm = jax.lax.rsqrt(s2)
    a = jnp.abs(x) * inv_norm
    lg = jnp.log(a + 1e-12) * INV_LN2
    term = (x * inv_norm) * lg
    row_e = jnp.sum(term, axis=1, keepdims=True)

    t = t_col_ref[...]
    w = jnp.zeros((ROWS, 1), dtype=jnp.float32)
    for c in range(NUM_CLASSES):
        w = w + (t == c).astype(jnp.float32) * inv_ref[c]

    partial = jnp.sum(row_e * w)
    acc_ref[0] = acc_ref[0] - partial

    @pl.when(step == NSTEPS - 1)
    def _fin():
        out_ref[...] = jnp.full((1, 1), acc_ref[0], dtype=jnp.float32)


def kernel(ChannelNoiseMatixs, targets):
    targets = jnp.squeeze(targets)
    t_full = targets.reshape(16, B // 16)
    t_col = targets.reshape(B, 1)
    out = pl.pallas_call(
        _body,
        grid=(N_MATS, NB),
        in_specs=[
            pl.BlockSpec((16, B // 16), lambda i, j: (0, 0)),
            pl.BlockSpec((ROWS, 1), lambda i, j: (j, 0)),
            pl.BlockSpec((1, ROWS, D), lambda i, j: (i, j, 0)),
        ],
        out_specs=pl.BlockSpec((1, 1), lambda i, j: (0, 0)),
        out_shape=jax.ShapeDtypeStruct((1, 1), jnp.float32),
        scratch_shapes=[
            pltpu.SMEM((NUM_CLASSES,), jnp.float32),
            pltpu.SMEM((1,), jnp.float32),
        ],
    )(t_full, t_col, ChannelNoiseMatixs)
    return out[0, 0]


# speedup vs baseline: 1.3285x; 1.2833x over previous
import jax
import jax.numpy as jnp
from jax import lax
from jax.experimental import pallas as pl
from jax.experimental.pallas import tpu as pltpu

N_MATS = 4
B = 16384
D = 512
NUM_CLASSES = 10
ROWS = 2048
NB = B // ROWS
NSTEPS = N_MATS * NB


def _body(t_col_ref, x_ref, out_ref, acc_ref):
    i = pl.program_id(0)
    j = pl.program_id(1)
    step = i * NB + j

    @pl.when(step == 0)
    def _init():
        acc_ref[...] = jnp.zeros((8, 128), dtype=jnp.float32)

    x = x_ref[0]
    u = jnp.abs(x)
    l = jnp.log2(u + 1e-30)
    ones_col = jnp.ones((D, 1), dtype=jnp.float32)
    s1 = lax.dot_general(x * x, ones_col, (((1,), (0,)), ((), ())),
                         preferred_element_type=jnp.float32)
    s2 = lax.dot_general(x * l, ones_col, (((1,), (0,)), ((), ())),
                         preferred_element_type=jnp.float32)
    s3 = lax.dot_general(x, ones_col, (((1,), (0,)), ((), ())),
                         preferred_element_type=jnp.float32)
    e = lax.rsqrt(s1) * (0.5 * jnp.log2(s1) * s3 - s2)

    t = t_col_ref[...]
    onehot = (t == lax.broadcasted_iota(jnp.int32, (ROWS, 128), 1)
              ).astype(jnp.float32)
    eo = jnp.concatenate([e, jnp.ones_like(e)], axis=1)
    part = lax.dot_general(eo, onehot, (((0,), (0,)), ((), ())),
                           preferred_element_type=jnp.float32)
    acc_ref[0:2, :] = acc_ref[0:2, :] + part

    @pl.when(step == NSTEPS - 1)
    def _fin():
        csum = acc_ref[0:1, :]
        cnt = acc_ref[1:2, :] * (1.0 / N_MATS)
        present = cnt > 0
        c_present = jnp.sum(jnp.where(present, 1.0, 0.0))
        contrib = jnp.where(present, csum / jnp.where(present, cnt, 1.0), 0.0)
        total = jnp.sum(contrib) / (N_MATS * c_present)
        out_ref[...] = jnp.full((1, 1), total, dtype=jnp.float32)


def kernel(ChannelNoiseMatixs, targets):
    targets = jnp.squeeze(targets)
    t_col = targets.reshape(B, 1)
    out = pl.pallas_call(
        _body,
        grid=(N_MATS, NB),
        in_specs=[
            pl.BlockSpec((ROWS, 1), lambda i, j: (j, 0)),
            pl.BlockSpec((1, ROWS, D), lambda i, j: (i, j, 0)),
        ],
        out_specs=pl.BlockSpec((1, 1), lambda i, j: (0, 0)),
        out_shape=jax.ShapeDtypeStruct((1, 1), jnp.float32),
        scratch_shapes=[
            pltpu.VMEM((8, 128), jnp.float32),
        ],
    )(t_col, ChannelNoiseMatixs)
    return out[0, 0]


# speedup vs baseline: 1.4688x; 1.1056x over previous
import jax
import jax.numpy as jnp
from jax import lax
from jax.experimental import pallas as pl
from jax.experimental.pallas import tpu as pltpu

N_MATS = 4
B = 16384
D = 512
NUM_CLASSES = 10
ROWS = 2048
NB = B // ROWS
NSTEPS = N_MATS * NB


def _body(t_col_ref, x_ref, out_ref, acc_ref, oh_ref):
    j = pl.program_id(0)
    i = pl.program_id(1)
    step = j * N_MATS + i

    @pl.when(step == 0)
    def _init():
        acc_ref[...] = jnp.zeros((8, 128), dtype=jnp.float32)

    @pl.when(i == 0)
    def _mkoh():
        t = t_col_ref[...]
        oh = (t == lax.broadcasted_iota(jnp.int32, (ROWS, 128), 1)
              ).astype(jnp.float32)
        oh_ref[...] = oh
        acc_ref[1:2, :] = acc_ref[1:2, :] + jnp.sum(oh, axis=0, keepdims=True)

    x = x_ref[...]
    sq = x * x
    l = jnp.log2(sq + 1e-38)
    ones_col = jnp.ones((D, 1), dtype=jnp.float32)
    s1 = jnp.sum(sq, axis=1, keepdims=True)
    s3 = jnp.sum(x, axis=1, keepdims=True)
    s2 = lax.dot_general(x * l, ones_col, (((1,), (0,)), ((), ())),
                         preferred_element_type=jnp.float32)
    e = lax.rsqrt(s1) * (jnp.log2(s1) * s3 - s2)
    part = lax.dot_general(e, oh_ref[...], (((0,), (0,)), ((), ())),
                           preferred_element_type=jnp.float32)
    acc_ref[0:1, :] = acc_ref[0:1, :] + part

    @pl.when(step == NSTEPS - 1)
    def _fin():
        csum = acc_ref[0:1, :]
        cnt = acc_ref[1:2, :]
        present = cnt > 0
        c_present = jnp.sum(jnp.where(present, 1.0, 0.0))
        contrib = jnp.where(present, csum / jnp.where(present, cnt, 1.0), 0.0)
        total = jnp.sum(contrib) / (2.0 * N_MATS * c_present)
        out_ref[...] = jnp.full((1, 1), total, dtype=jnp.float32)


def kernel(ChannelNoiseMatixs, targets):
    targets = jnp.squeeze(targets)
    t_col = targets.reshape(B, 1)
    out = pl.pallas_call(
        _body,
        grid=(NB, N_MATS),
        in_specs=[
            pl.BlockSpec((ROWS, 1), lambda j, i: (j, 0)),
            pl.BlockSpec((ROWS, D), lambda j, i: (i * NB + j, 0)),
        ],
        out_specs=pl.BlockSpec((1, 1), lambda j, i: (0, 0)),
        out_shape=jax.ShapeDtypeStruct((1, 1), jnp.float32),
        scratch_shapes=[
            pltpu.VMEM((8, 128), jnp.float32),
            pltpu.VMEM((ROWS, 128), jnp.float32),
        ],
    )(t_col, ChannelNoiseMatixs.reshape(N_MATS * B, D))
    return out[0, 0]


# speedup vs baseline: 1.5967x; 1.0871x over previous
import jax
import jax.numpy as jnp
from jax import lax
from jax.experimental import pallas as pl
from jax.experimental.pallas import tpu as pltpu

N_MATS = 4
B = 16384
D = 512
NUM_CLASSES = 10
ROWS = 4096
NB = B // ROWS
NSTEPS = N_MATS * NB


def _body(t_col_ref, x_ref, out_ref, acc_ref, oh_ref):
    j = pl.program_id(0)
    i = pl.program_id(1)
    step = j * N_MATS + i

    @pl.when(step == 0)
    def _init():
        acc_ref[...] = jnp.zeros((8, 128), dtype=jnp.float32)

    @pl.when(i == 0)
    def _mkoh():
        t = t_col_ref[...]
        oh = (t == lax.broadcasted_iota(jnp.int32, (ROWS, 128), 1)
              ).astype(jnp.float32)
        oh_ref[...] = oh
        acc_ref[1:2, :] = acc_ref[1:2, :] + jnp.sum(oh, axis=0, keepdims=True)

    x = x_ref[...]
    sq = x * x
    l = jnp.log2(sq + 1e-38)
    ones_col = jnp.ones((D, 1), dtype=jnp.float32)
    s1 = jnp.sum(sq, axis=1, keepdims=True)
    s3 = jnp.sum(x, axis=1, keepdims=True)
    s2 = lax.dot_general(x * l, ones_col, (((1,), (0,)), ((), ())),
                         preferred_element_type=jnp.float32)
    e = lax.rsqrt(s1) * (jnp.log2(s1) * s3 - s2)
    part = lax.dot_general(e, oh_ref[...], (((0,), (0,)), ((), ())),
                           preferred_element_type=jnp.float32)
    acc_ref[0:1, :] = acc_ref[0:1, :] + part

    @pl.when(step == NSTEPS - 1)
    def _fin():
        csum = acc_ref[0:1, :]
        cnt = acc_ref[1:2, :]
        present = cnt > 0
        c_present = jnp.sum(jnp.where(present, 1.0, 0.0))
        contrib = jnp.where(present, csum / jnp.where(present, cnt, 1.0), 0.0)
        total = jnp.sum(contrib) / (2.0 * N_MATS * c_present)
        out_ref[...] = jnp.full((1, 1), total, dtype=jnp.float32)


def kernel(ChannelNoiseMatixs, targets):
    targets = jnp.squeeze(targets)
    t_col = targets.reshape(B, 1)
    out = pl.pallas_call(
        _body,
        grid=(NB, N_MATS),
        in_specs=[
            pl.BlockSpec((ROWS, 1), lambda j, i: (j, 0)),
            pl.BlockSpec((ROWS, D), lambda j, i: (i * NB + j, 0)),
        ],
        out_specs=pl.BlockSpec((1, 1), lambda j, i: (0, 0)),
        out_shape=jax.ShapeDtypeStruct((1, 1), jnp.float32),
        scratch_shapes=[
            pltpu.VMEM((8, 128), jnp.float32),
            pltpu.VMEM((ROWS, 128), jnp.float32),
        ],
    )(t_col, ChannelNoiseMatixs.reshape(N_MATS * B, D))
    return out[0, 0]
